# Initial kernel scaffold; baseline (speedup 1.0000x reference)
import jax
import jax.numpy as jnp
from jax.experimental import pallas as pl


def kernel(y_pred, y):
    raise NotImplementedError("write your pallas kernel here")


# speedup vs baseline: 1.7645x; 1.7645x over previous
import jax
import jax.numpy as jnp
from jax.experimental import pallas as pl
from jax.experimental.pallas import tpu as pltpu

K = 5
B = 128
N = 100000
BLK = 12544
NB = (N + BLK - 1) // BLK


def _acc_body(y_vec_ref, x_ref, out_ref, v_ref, cnt_ref):
    phase = pl.program_id(0)
    j = pl.program_id(1)

    yv = y_vec_ref[...]
    x = x_ref[...]
    col = j * BLK + jax.lax.broadcasted_iota(jnp.int32, (B, BLK), 1)

    @pl.when(jnp.logical_and(phase == 0, j == 0))
    def _init():
        v_ref[...] = jnp.full((B, 1), -jnp.inf, jnp.float32)
        cnt_ref[...] = jnp.zeros((B, 1), jnp.float32)

    @pl.when(phase == 0)
    def _extract():
        m = jnp.where(col == yv, x, -jnp.inf)
        v_ref[...] = jnp.maximum(v_ref[...], jnp.max(m, axis=1, keepdims=True))

    @pl.when(phase == 1)
    def _count():
        v = v_ref[...]
        valid = col < N
        gt = jnp.logical_and(x > v, valid)
        eq_lt = jnp.logical_and(x == v, col < yv)
        hits = jnp.logical_or(gt, eq_lt).astype(jnp.float32)
        cnt_ref[...] += jnp.sum(hits, axis=1, keepdims=True)

    @pl.when(jnp.logical_and(phase == 1, j == NB - 1))
    def _finalize():
        in_topk = (cnt_ref[...] < float(K)).astype(jnp.int32)
        out_ref[0, 0] = jnp.sum(in_topk)


def kernel(y_pred, y):
    y_vec = y.astype(jnp.int32).reshape(B, 1)
    out = pl.pallas_call(
        _acc_body,
        grid=(2, NB),
        in_specs=[
            pl.BlockSpec((B, 1), lambda p, j: (0, 0)),
            pl.BlockSpec((B, BLK), lambda p, j: (0, j)),
        ],
        out_specs=pl.BlockSpec(memory_space=pltpu.SMEM),
        out_shape=jax.ShapeDtypeStruct((1, 1), jnp.int32),
        scratch_shapes=[
            pltpu.VMEM((B, 1), jnp.float32),
            pltpu.VMEM((B, 1), jnp.float32),
        ],
        compiler_params=pltpu.CompilerParams(
            dimension_semantics=("arbitrary", "arbitrary"),
        ),
    )(y_vec, y_pred)
    return out.reshape(())


# speedup vs baseline: 2.1711x; 1.2304x over previous
import jax
import jax.numpy as jnp
from jax.experimental import pallas as pl
from jax.experimental.pallas import tpu as pltpu

K = 5
B = 128
N = 100000
BLK = 12544
NB = (N + BLK - 1) // BLK
C0 = (NB - 1) * BLK
MAX_OFF = (N - 128) // 128 * 128


def _body(y_smem, y_vec_ref, x_ref, ypred_hbm, out_ref, gbuf, v_ref, cnt_ref, sem):
    j = pl.program_id(0)
    blk = jax.lax.rem(j + NB - 1, NB)
    yv = y_vec_ref[...]
    x = x_ref[...]
    col = blk * BLK + jax.lax.broadcasted_iota(jnp.int32, (B, BLK), 1)

    @pl.when(j == 0)
    def _first():
        def _start(i, _):
            off = jnp.minimum((y_smem[i] // 128) * 128, MAX_OFF)
            row0 = (i // 8) * 8
            pltpu.make_async_copy(
                ypred_hbm.at[pl.ds(row0, 8), pl.ds(off, 128)],
                gbuf.at[i],
                sem,
            ).start()
            return 0

        jax.lax.fori_loop(0, B, _start, 0)

        def _wait(i, _):
            pltpu.make_async_copy(
                ypred_hbm.at[pl.ds(0, 8), pl.ds(0, 128)],
                gbuf.at[0],
                sem,
            ).wait()
            return 0

        jax.lax.fori_loop(0, B, _wait, 0)

        off_vec = jnp.minimum((yv // 128) * 128, MAX_OFF)
        lane = (yv - off_vec).reshape(B, 1, 1)
        ri = jax.lax.rem(jax.lax.broadcasted_iota(jnp.int32, (B, 8, 128), 0), 8)
        si = jax.lax.broadcasted_iota(jnp.int32, (B, 8, 128), 1)
        li = jax.lax.broadcasted_iota(jnp.int32, (B, 8, 128), 2)
        sel = jnp.logical_and(si == ri, li == lane)
        v_dma = jnp.sum(
            jnp.sum(jnp.where(sel, gbuf[...], 0.0), axis=2), axis=1, keepdims=True
        )
        v_blk = jnp.max(
            jnp.where(col == yv, x, -jnp.inf), axis=1, keepdims=True
        )
        v = jnp.where(yv >= C0, v_blk, v_dma)
        v_ref[...] = v

        xm = jnp.where(col < N, x, -jnp.inf)
        hits = jnp.logical_or(
            xm > v, jnp.logical_and(xm == v, col < yv)
        ).astype(jnp.float32)
        cnt_ref[...] = jnp.sum(hits, axis=1, keepdims=True)

    @pl.when(j > 0)
    def _count():
        v = v_ref[...]
        hits = jnp.logical_or(
            x > v, jnp.logical_and(x == v, col < yv)
        ).astype(jnp.float32)
        cnt_ref[...] += jnp.sum(hits, axis=1, keepdims=True)

    @pl.when(j == NB - 1)
    def _finalize():
        in_topk = (cnt_ref[...] < float(K)).astype(jnp.int32)
        out_ref[0, 0] = jnp.sum(in_topk)


def kernel(y_pred, y):
    y32 = y.astype(jnp.int32)
    y_vec = y32.reshape(B, 1)
    grid_spec = pltpu.PrefetchScalarGridSpec(
        num_scalar_prefetch=1,
        grid=(NB,),
        in_specs=[
            pl.BlockSpec((B, 1), lambda j, y_s: (0, 0)),
            pl.BlockSpec((B, BLK), lambda j, y_s: (0, (j + NB - 1) % NB)),
            pl.BlockSpec(memory_space=pltpu.MemorySpace.HBM),
        ],
        out_specs=pl.BlockSpec(memory_space=pltpu.MemorySpace.SMEM),
        scratch_shapes=[
            pltpu.VMEM((B, 8, 128), jnp.float32),
            pltpu.VMEM((B, 1), jnp.float32),
            pltpu.VMEM((B, 1), jnp.float32),
            pltpu.SemaphoreType.DMA,
        ],
    )
    out = pl.pallas_call(
        _body,
        grid_spec=grid_spec,
        out_shape=jax.ShapeDtypeStruct((1, 1), jnp.int32),
        compiler_params=pltpu.CompilerParams(
            dimension_semantics=("arbitrary",),
        ),
    )(y32, y_vec, y_pred, y_pred)
    return out.reshape(())
